# baseline (speedup 1.0000x reference)
import functools

import jax
import jax.numpy as jnp
from jax import lax
from jax.experimental import pallas as pl
from jax.experimental.pallas import tpu as pltpu
from jax.experimental.pallas import tpu_sc as plsc

DIM = 64
B = 16384
NC = 2
NS = 16
NW = NC * NS
BPW = B // NW
CHUNK = 128
NCHUNK = BPW // CHUNK
NPASS = 2
CPP = NCHUNK // NPASS
PP = BPW // NPASS
LANES = 16
GPC = CHUNK // LANES


def _sc_body(gc_hbm, oc_hbm, gx_hbm, ox_hbm, t_hbm, out_hbm,
             idx_c, idx_x, off_c, off_x, rows_c, rows_x, outv, sem):
    c = lax.axis_index("c")
    s = lax.axis_index("s")
    wid = s * NC + c
    base = wid * NCHUNK

    pltpu.sync_copy(gc_hbm.at[pl.ds(base, NCHUNK)], idx_c)
    pltpu.sync_copy(oc_hbm.at[pl.ds(base, NCHUNK)], off_c)
    pltpu.sync_copy(gx_hbm.at[pl.ds(base, NCHUNK)], idx_x)
    pltpu.sync_copy(ox_hbm.at[pl.ds(base, NCHUNK)], off_x)

    iot = lax.iota(jnp.int32, LANES)
    acc = jnp.zeros((LANES,), jnp.float32)

    for p in range(NPASS):
        copies = []
        for j in range(CPP):
            copies.append(pltpu.async_copy(
                t_hbm.at[idx_c.at[p * CPP + j]],
                rows_c.at[pl.ds(j * CHUNK, CHUNK)], sem))
            copies.append(pltpu.async_copy(
                t_hbm.at[idx_x.at[p * CPP + j]],
                rows_x.at[pl.ds(j * CHUNK, CHUNK)], sem))
        for cp in copies:
            cp.wait()

        for j in range(CPP):
            for h in range(GPC):
                rvec = j * CHUNK + h * LANES + iot
                ocv = off_c[p * CPP + j, pl.ds(h * LANES, LANES)]
                oxv = off_x[p * CPP + j, pl.ds(h * LANES, LANES)]

                def body(k, a, rvec=rvec, ocv=ocv, oxv=oxv):
                    cv = plsc.load_gather(rows_c, [rvec, ocv + k])
                    xv = plsc.load_gather(rows_x, [rvec, oxv + k])
                    return a + cv * xv

                acc = lax.fori_loop(0, DIM, body, acc)

    outv[...] = acc
    pltpu.sync_copy(outv, out_hbm.at[wid])


_sc_call = functools.partial(
    pl.kernel,
    mesh=plsc.VectorSubcoreMesh(core_axis_name="c", subcore_axis_name="s"),
    out_type=jax.ShapeDtypeStruct((NW, LANES), jnp.float32),
    scratch_types=[
        pltpu.VMEM((NCHUNK, CHUNK), jnp.int32),
        pltpu.VMEM((NCHUNK, CHUNK), jnp.int32),
        pltpu.VMEM((NCHUNK, CHUNK), jnp.int32),
        pltpu.VMEM((NCHUNK, CHUNK), jnp.int32),
        pltpu.VMEM((PP, CHUNK), jnp.float32),
        pltpu.VMEM((PP, CHUNK), jnp.float32),
        pltpu.VMEM((LANES,), jnp.float32),
        pltpu.SemaphoreType.DMA,
    ],
    compiler_params=pltpu.CompilerParams(needs_layout_passes=False),
)(_sc_body)


@jax.jit
def kernel(center_words, context_words, embeddings):
    table = embeddings.reshape(500000, 128)
    cw = jnp.asarray(center_words, jnp.int32)
    xw = jnp.asarray(context_words, jnp.int32)
    shape2 = (NW * NCHUNK, CHUNK)
    gc = (cw >> 1).reshape(shape2)
    oc = ((cw & 1) << 6).reshape(shape2)
    gx = (xw >> 1).reshape(shape2)
    ox = ((xw & 1) << 6).reshape(shape2)
    partials = _sc_call(gc, oc, gx, ox, table)
    return jnp.sum(partials)


# speedup vs baseline: 1.0195x; 1.0195x over previous
import functools

import jax
import jax.numpy as jnp
from jax import lax
from jax.experimental import pallas as pl
from jax.experimental.pallas import tpu as pltpu
from jax.experimental.pallas import tpu_sc as plsc

DIM = 64
B = 16384
NC = 2
NS = 16
NW = NC * NS
BPW = B // NW
CHUNK = 128
NCHUNK = BPW // CHUNK
NPASS = 2
CPP = NCHUNK // NPASS
PP = BPW // NPASS
LANES = 16
GPC = CHUNK // LANES


def _sc_body(gc_hbm, oc_hbm, gx_hbm, ox_hbm, t_hbm, out_hbm,
             idx_c, idx_x, off_c, off_x, rows_c, rows_x, outv, sem):
    c = lax.axis_index("c")
    s = lax.axis_index("s")
    wid = s * NC + c
    base = wid * NCHUNK

    pltpu.sync_copy(gc_hbm.at[pl.ds(base, NCHUNK)], idx_c)
    pltpu.sync_copy(oc_hbm.at[pl.ds(base, NCHUNK)], off_c)
    pltpu.sync_copy(gx_hbm.at[pl.ds(base, NCHUNK)], idx_x)
    pltpu.sync_copy(ox_hbm.at[pl.ds(base, NCHUNK)], off_x)

    iot = lax.iota(jnp.int32, LANES)
    zero = jnp.zeros((LANES,), jnp.float32)
    accs = (zero, zero, zero, zero)

    for p in range(NPASS):
        copies = []
        for j in range(CPP):
            copies.append(pltpu.async_copy(
                t_hbm.at[idx_c.at[p * CPP + j]],
                rows_c.at[pl.ds(j * CHUNK, CHUNK)], sem))
            copies.append(pltpu.async_copy(
                t_hbm.at[idx_x.at[p * CPP + j]],
                rows_x.at[pl.ds(j * CHUNK, CHUNK)], sem))
        for cp in copies:
            cp.wait()

        for j in range(CPP):
            for h in range(GPC):
                rvec = j * CHUNK + h * LANES + iot
                ocv = off_c[p * CPP + j, pl.ds(h * LANES, LANES)]
                oxv = off_x[p * CPP + j, pl.ds(h * LANES, LANES)]

                def body(k4, a, rvec=rvec, ocv=ocv, oxv=oxv):
                    out = []
                    for u in range(4):
                        k = k4 * 4 + u
                        cv = plsc.load_gather(rows_c, [rvec, ocv + k])
                        xv = plsc.load_gather(rows_x, [rvec, oxv + k])
                        out.append(a[u] + cv * xv)
                    return tuple(out)

                accs = lax.fori_loop(0, DIM // 4, body, accs)

    outv[...] = (accs[0] + accs[1]) + (accs[2] + accs[3])
    pltpu.sync_copy(outv, out_hbm.at[wid])


_sc_call = functools.partial(
    pl.kernel,
    mesh=plsc.VectorSubcoreMesh(core_axis_name="c", subcore_axis_name="s"),
    out_type=jax.ShapeDtypeStruct((NW, LANES), jnp.float32),
    scratch_types=[
        pltpu.VMEM((NCHUNK, CHUNK), jnp.int32),
        pltpu.VMEM((NCHUNK, CHUNK), jnp.int32),
        pltpu.VMEM((NCHUNK, CHUNK), jnp.int32),
        pltpu.VMEM((NCHUNK, CHUNK), jnp.int32),
        pltpu.VMEM((PP, CHUNK), jnp.float32),
        pltpu.VMEM((PP, CHUNK), jnp.float32),
        pltpu.VMEM((LANES,), jnp.float32),
        pltpu.SemaphoreType.DMA,
    ],
    compiler_params=pltpu.CompilerParams(
        use_tc_tiling_on_sc=True, needs_layout_passes=False),
)(_sc_body)


@jax.jit
def kernel(center_words, context_words, embeddings):
    table = embeddings.reshape(500000, 128)
    cw = jnp.asarray(center_words, jnp.int32)
    xw = jnp.asarray(context_words, jnp.int32)
    shape2 = (NW * NCHUNK, CHUNK)
    gc = (cw >> 1).reshape(shape2)
    oc = ((cw & 1) << 6).reshape(shape2)
    gx = (xw >> 1).reshape(shape2)
    ox = ((xw & 1) << 6).reshape(shape2)
    partials = _sc_call(gc, oc, gx, ox, table)
    return jnp.sum(partials)


# speedup vs baseline: 1.2500x; 1.2261x over previous
import functools

import jax
import jax.numpy as jnp
from jax import lax
from jax.experimental import pallas as pl
from jax.experimental.pallas import tpu as pltpu
from jax.experimental.pallas import tpu_sc as plsc

VOCAB_ROWS = 1000000
DIM = 64
B = 16384
NC = 2
NS = 16
NW = NC * NS
BPW = B // NW
CHUNK = 128
NCHUNK = BPW // CHUNK
NPASS = 2
CPP = NCHUNK // NPASS
PP = BPW // NPASS
LANES = 16
GPC = CHUNK // LANES

CB = 2048
NBLK = (VOCAB_ROWS + CB - 1) // CB
SROWS = VOCAB_ROWS // 2


def _repack_body(t_ref, out_ref, tt_ref):
    tt_ref[...] = t_ref[...].T
    out_ref[:, 0:DIM] = tt_ref[pl.Slice(0, CB // 2, 2), :]
    out_ref[:, DIM:2 * DIM] = tt_ref[pl.Slice(1, CB // 2, 2), :]


_repack = pl.pallas_call(
    _repack_body,
    grid=(NBLK,),
    in_specs=[pl.BlockSpec((DIM, CB), lambda i: (0, i))],
    out_specs=pl.BlockSpec((CB // 2, 128), lambda i: (i, 0)),
    out_shape=jax.ShapeDtypeStruct((SROWS, 128), jnp.float32),
    scratch_shapes=[pltpu.VMEM((CB, DIM), jnp.float32)],
)


def _sc_body(gc_hbm, oc_hbm, gx_hbm, ox_hbm, t_hbm, out_hbm,
             idx_c, idx_x, off_c, off_x, rows_c, rows_x, outv, sem):
    c = lax.axis_index("c")
    s = lax.axis_index("s")
    wid = s * NC + c
    base = wid * NCHUNK

    pltpu.sync_copy(gc_hbm.at[pl.ds(base, NCHUNK)], idx_c)
    pltpu.sync_copy(oc_hbm.at[pl.ds(base, NCHUNK)], off_c)
    pltpu.sync_copy(gx_hbm.at[pl.ds(base, NCHUNK)], idx_x)
    pltpu.sync_copy(ox_hbm.at[pl.ds(base, NCHUNK)], off_x)

    iot = lax.iota(jnp.int32, LANES)
    zero = jnp.zeros((LANES,), jnp.float32)
    accs = (zero, zero, zero, zero)

    for p in range(NPASS):
        copies = []
        for j in range(CPP):
            copies.append(pltpu.async_copy(
                t_hbm.at[idx_c.at[p * CPP + j]],
                rows_c.at[pl.ds(j * CHUNK, CHUNK)], sem))
            copies.append(pltpu.async_copy(
                t_hbm.at[idx_x.at[p * CPP + j]],
                rows_x.at[pl.ds(j * CHUNK, CHUNK)], sem))
        for cp in copies:
            cp.wait()

        for j in range(CPP):
            for h in range(GPC):
                rvec = j * CHUNK + h * LANES + iot
                ocv = off_c[p * CPP + j, pl.ds(h * LANES, LANES)]
                oxv = off_x[p * CPP + j, pl.ds(h * LANES, LANES)]

                def body(k4, a, rvec=rvec, ocv=ocv, oxv=oxv):
                    out = []
                    for u in range(4):
                        k = k4 * 4 + u
                        cv = plsc.load_gather(rows_c, [rvec, ocv + k])
                        xv = plsc.load_gather(rows_x, [rvec, oxv + k])
                        out.append(a[u] + cv * xv)
                    return tuple(out)

                accs = lax.fori_loop(0, DIM // 4, body, accs)

    outv[...] = (accs[0] + accs[1]) + (accs[2] + accs[3])
    pltpu.sync_copy(outv, out_hbm.at[wid])


_sc_call = functools.partial(
    pl.kernel,
    mesh=plsc.VectorSubcoreMesh(core_axis_name="c", subcore_axis_name="s"),
    out_type=jax.ShapeDtypeStruct((NW, LANES), jnp.float32),
    scratch_types=[
        pltpu.VMEM((NCHUNK, CHUNK), jnp.int32),
        pltpu.VMEM((NCHUNK, CHUNK), jnp.int32),
        pltpu.VMEM((NCHUNK, CHUNK), jnp.int32),
        pltpu.VMEM((NCHUNK, CHUNK), jnp.int32),
        pltpu.VMEM((PP, CHUNK), jnp.float32),
        pltpu.VMEM((PP, CHUNK), jnp.float32),
        pltpu.VMEM((LANES,), jnp.float32),
        pltpu.SemaphoreType.DMA,
    ],
    compiler_params=pltpu.CompilerParams(
        use_tc_tiling_on_sc=True, needs_layout_passes=False),
)(_sc_body)


@jax.jit
def kernel(center_words, context_words, embeddings):
    table = _repack(embeddings.T)
    cw = jnp.asarray(center_words, jnp.int32)
    xw = jnp.asarray(context_words, jnp.int32)
    shape2 = (NW * NCHUNK, CHUNK)
    gc = (cw >> 1).reshape(shape2)
    oc = ((cw & 1) << 6).reshape(shape2)
    gx = (xw >> 1).reshape(shape2)
    ox = ((xw & 1) << 6).reshape(shape2)
    partials = _sc_call(gc, oc, gx, ox, table)
    return jnp.sum(partials)


# speedup vs baseline: 1.2999x; 1.0399x over previous
import functools

import jax
import jax.numpy as jnp
from jax import lax
from jax.experimental import pallas as pl
from jax.experimental.pallas import tpu as pltpu
from jax.experimental.pallas import tpu_sc as plsc

VOCAB_ROWS = 1000000
DIM = 64
B = 16384
NC = 2
NS = 16
NW = NC * NS
BPW = B // NW
CHUNK = 128
NCHUNK = BPW // CHUNK
NPASS = 2
CPP = NCHUNK // NPASS
PP = BPW // NPASS
LANES = 16
GPC = CHUNK // LANES

CB = 2048
HB = CB // 2
NBLK = (VOCAB_ROWS + CB - 1) // CB
SROWS = NBLK * HB


def _repack_body(t_ref, out_ref):
    t = t_ref[...]
    i64 = lax.broadcasted_iota(jnp.int32, (DIM, DIM), 0)
    eye = (i64 == i64.T).astype(jnp.float32)
    tt = lax.dot_general(t, eye, (((0,), (0,)), ((), ())),
                         preferred_element_type=jnp.float32)
    out_ref[:, 0:DIM] = tt[0:HB]
    out_ref[:, DIM:2 * DIM] = tt[HB:CB]


_repack = pl.pallas_call(
    _repack_body,
    grid=(NBLK,),
    in_specs=[pl.BlockSpec((DIM, CB), lambda i: (0, i))],
    out_specs=pl.BlockSpec((HB, 128), lambda i: (i, 0)),
    out_shape=jax.ShapeDtypeStruct((SROWS, 128), jnp.float32),
)


def _sc_body(gc_hbm, oc_hbm, gx_hbm, ox_hbm, t_hbm, out_hbm,
             idx_c, idx_x, off_c, off_x, rows_c, rows_x, outv, sem):
    c = lax.axis_index("c")
    s = lax.axis_index("s")
    wid = s * NC + c
    base = wid * NCHUNK

    pltpu.sync_copy(gc_hbm.at[pl.ds(base, NCHUNK)], idx_c)
    pltpu.sync_copy(oc_hbm.at[pl.ds(base, NCHUNK)], off_c)
    pltpu.sync_copy(gx_hbm.at[pl.ds(base, NCHUNK)], idx_x)
    pltpu.sync_copy(ox_hbm.at[pl.ds(base, NCHUNK)], off_x)

    iot = lax.iota(jnp.int32, LANES)
    zero = jnp.zeros((LANES,), jnp.float32)
    accs = (zero, zero, zero, zero)

    for p in range(NPASS):
        copies = []
        for j in range(CPP):
            copies.append(pltpu.async_copy(
                t_hbm.at[idx_c.at[p * CPP + j]],
                rows_c.at[pl.ds(j * CHUNK, CHUNK)], sem))
            copies.append(pltpu.async_copy(
                t_hbm.at[idx_x.at[p * CPP + j]],
                rows_x.at[pl.ds(j * CHUNK, CHUNK)], sem))
        for cp in copies:
            cp.wait()

        for j in range(CPP):
            for h in range(GPC):
                rvec = j * CHUNK + h * LANES + iot
                ocv = off_c[p * CPP + j, pl.ds(h * LANES, LANES)]
                oxv = off_x[p * CPP + j, pl.ds(h * LANES, LANES)]

                def body(k4, a, rvec=rvec, ocv=ocv, oxv=oxv):
                    out = []
                    for u in range(4):
                        k = k4 * 4 + u
                        cv = plsc.load_gather(rows_c, [rvec, ocv + k])
                        xv = plsc.load_gather(rows_x, [rvec, oxv + k])
                        out.append(a[u] + cv * xv)
                    return tuple(out)

                accs = lax.fori_loop(0, DIM // 4, body, accs)

    outv[...] = (accs[0] + accs[1]) + (accs[2] + accs[3])
    pltpu.sync_copy(outv, out_hbm.at[wid])


_sc_call = functools.partial(
    pl.kernel,
    mesh=plsc.VectorSubcoreMesh(core_axis_name="c", subcore_axis_name="s"),
    out_type=jax.ShapeDtypeStruct((NW, LANES), jnp.float32),
    scratch_types=[
        pltpu.VMEM((NCHUNK, CHUNK), jnp.int32),
        pltpu.VMEM((NCHUNK, CHUNK), jnp.int32),
        pltpu.VMEM((NCHUNK, CHUNK), jnp.int32),
        pltpu.VMEM((NCHUNK, CHUNK), jnp.int32),
        pltpu.VMEM((PP, CHUNK), jnp.float32),
        pltpu.VMEM((PP, CHUNK), jnp.float32),
        pltpu.VMEM((LANES,), jnp.float32),
        pltpu.SemaphoreType.DMA,
    ],
    compiler_params=pltpu.CompilerParams(
        use_tc_tiling_on_sc=True, needs_layout_passes=False),
)(_sc_body)


@jax.jit
def kernel(center_words, context_words, embeddings):
    table = _repack(embeddings.T)
    cw = jnp.asarray(center_words, jnp.int32)
    xw = jnp.asarray(context_words, jnp.int32)
    shape2 = (NW * NCHUNK, CHUNK)
    def sup(r):
        return ((r >> 11) << 10) | (r & (HB - 1))

    def off(r):
        return ((r >> 10) & 1) << 6

    gc = sup(cw).reshape(shape2)
    oc = off(cw).reshape(shape2)
    gx = sup(xw).reshape(shape2)
    ox = off(xw).reshape(shape2)
    partials = _sc_call(gc, oc, gx, ox, table)
    return jnp.sum(partials)


# speedup vs baseline: 1.7490x; 1.3455x over previous
import functools

import jax
import jax.numpy as jnp
from jax import lax
from jax.experimental import pallas as pl
from jax.experimental.pallas import tpu as pltpu
from jax.experimental.pallas import tpu_sc as plsc

VOCAB_ROWS = 1000000
DIM = 64
B = 16384
NC = 2
NS = 16
NW = NC * NS
BPW = B // NW
CHUNK = 128
NCHUNK = BPW // CHUNK
NPASS = 2
CPP = NCHUNK // NPASS
PP = BPW // NPASS
LANES = 16
GPC = CHUNK // LANES

CB = 4096
HB = CB // 2
NBLK = (VOCAB_ROWS + CB - 1) // CB
SROWS = NBLK * HB


def _repack_body(t_ref, out_ref):
    t = t_ref[...]
    i64 = lax.broadcasted_iota(jnp.int32, (DIM, DIM), 0)
    eye = (i64 == i64.T).astype(jnp.float32)
    tt = lax.dot_general(t, eye, (((0,), (0,)), ((), ())),
                         preferred_element_type=jnp.float32)
    out_ref[:, 0:DIM] = tt[0:HB]
    out_ref[:, DIM:2 * DIM] = tt[HB:CB]


_repack = pl.pallas_call(
    _repack_body,
    grid=(NBLK,),
    in_specs=[pl.BlockSpec((DIM, CB), lambda i: (0, i))],
    out_specs=pl.BlockSpec((HB, 128), lambda i: (i, 0)),
    out_shape=jax.ShapeDtypeStruct((SROWS, 128), jnp.float32),
)


def _sc_body(gc_hbm, oc_hbm, gx_hbm, ox_hbm, t_hbm, out_hbm,
             idx_c, idx_x, off_c, off_x, rows_c, rows_x, outv, sem):
    c = lax.axis_index("c")
    s = lax.axis_index("s")
    wid = s * NC + c
    base = wid * NCHUNK

    pltpu.sync_copy(gc_hbm.at[pl.ds(base, NCHUNK)], idx_c)
    pltpu.sync_copy(oc_hbm.at[pl.ds(base, NCHUNK)], off_c)
    pltpu.sync_copy(gx_hbm.at[pl.ds(base, NCHUNK)], idx_x)
    pltpu.sync_copy(ox_hbm.at[pl.ds(base, NCHUNK)], off_x)

    iot = lax.iota(jnp.int32, LANES)
    zero = jnp.zeros((LANES,), jnp.float32)
    accs = (zero, zero, zero, zero)

    for p in range(NPASS):
        copies = []
        for j in range(CPP):
            copies.append(pltpu.async_copy(
                t_hbm.at[idx_c.at[p * CPP + j]],
                rows_c.at[pl.ds(j * CHUNK, CHUNK)], sem))
            copies.append(pltpu.async_copy(
                t_hbm.at[idx_x.at[p * CPP + j]],
                rows_x.at[pl.ds(j * CHUNK, CHUNK)], sem))
        for cp in copies:
            cp.wait()

        for j in range(CPP):
            for h in range(GPC):
                rvec = j * CHUNK + h * LANES + iot
                ocv = off_c[p * CPP + j, pl.ds(h * LANES, LANES)]
                oxv = off_x[p * CPP + j, pl.ds(h * LANES, LANES)]

                def body(k4, a, rvec=rvec, ocv=ocv, oxv=oxv):
                    out = []
                    for u in range(4):
                        k = k4 * 4 + u
                        cv = plsc.load_gather(rows_c, [rvec, ocv + k])
                        xv = plsc.load_gather(rows_x, [rvec, oxv + k])
                        out.append(a[u] + cv * xv)
                    return tuple(out)

                accs = lax.fori_loop(0, DIM // 4, body, accs)

    outv[...] = (accs[0] + accs[1]) + (accs[2] + accs[3])
    pltpu.sync_copy(outv, out_hbm.at[wid])


_sc_call = functools.partial(
    pl.kernel,
    mesh=plsc.VectorSubcoreMesh(core_axis_name="c", subcore_axis_name="s"),
    out_type=jax.ShapeDtypeStruct((NW, LANES), jnp.float32),
    scratch_types=[
        pltpu.VMEM((NCHUNK, CHUNK), jnp.int32),
        pltpu.VMEM((NCHUNK, CHUNK), jnp.int32),
        pltpu.VMEM((NCHUNK, CHUNK), jnp.int32),
        pltpu.VMEM((NCHUNK, CHUNK), jnp.int32),
        pltpu.VMEM((PP, CHUNK), jnp.float32),
        pltpu.VMEM((PP, CHUNK), jnp.float32),
        pltpu.VMEM((LANES,), jnp.float32),
        pltpu.SemaphoreType.DMA,
    ],
    compiler_params=pltpu.CompilerParams(
        use_tc_tiling_on_sc=True, needs_layout_passes=False),
)(_sc_body)


@jax.jit
def kernel(center_words, context_words, embeddings):
    table = _repack(embeddings.T)
    cw = jnp.asarray(center_words, jnp.int32)
    xw = jnp.asarray(context_words, jnp.int32)
    shape2 = (NW * NCHUNK, CHUNK)
    cb_bits = CB.bit_length() - 1

    def sup(r):
        return ((r >> cb_bits) << (cb_bits - 1)) | (r & (HB - 1))

    def off(r):
        return ((r >> (cb_bits - 1)) & 1) << 6

    gc = sup(cw).reshape(shape2)
    oc = off(cw).reshape(shape2)
    gx = sup(xw).reshape(shape2)
    ox = off(xw).reshape(shape2)
    partials = _sc_call(gc, oc, gx, ox, table)
    return jnp.sum(partials)


# speedup vs baseline: 1.7550x; 1.0034x over previous
import functools

import jax
import jax.numpy as jnp
from jax import lax
from jax.experimental import pallas as pl
from jax.experimental.pallas import tpu as pltpu
from jax.experimental.pallas import tpu_sc as plsc

VOCAB_ROWS = 1000000
DIM = 64
B = 16384
NC = 2
NS = 16
NW = NC * NS
BPW = B // NW
CHUNK = 128
NCHUNK = BPW // CHUNK
NPASS = 2
CPP = NCHUNK // NPASS
PP = BPW // NPASS
LANES = 16
GPC = CHUNK // LANES

CB = 4096
HB = CB // 2
NBLK = (VOCAB_ROWS + CB - 1) // CB
SROWS = NBLK * HB


def _repack_body(t_ref, out_ref):
    tt = t_ref[...].T
    out_ref[:, 0:DIM] = tt[0:HB]
    out_ref[:, DIM:2 * DIM] = tt[HB:CB]


_repack = pl.pallas_call(
    _repack_body,
    grid=(NBLK,),
    in_specs=[pl.BlockSpec((DIM, CB), lambda i: (0, i))],
    out_specs=pl.BlockSpec((HB, 128), lambda i: (i, 0)),
    out_shape=jax.ShapeDtypeStruct((SROWS, 128), jnp.float32),
)


def _sc_body(gc_hbm, oc_hbm, gx_hbm, ox_hbm, t_hbm, out_hbm,
             idx_c, idx_x, off_c, off_x, rows_c, rows_x, outv, sem):
    c = lax.axis_index("c")
    s = lax.axis_index("s")
    wid = s * NC + c
    base = wid * NCHUNK

    pltpu.sync_copy(gc_hbm.at[pl.ds(base, NCHUNK)], idx_c)
    pltpu.sync_copy(oc_hbm.at[pl.ds(base, NCHUNK)], off_c)
    pltpu.sync_copy(gx_hbm.at[pl.ds(base, NCHUNK)], idx_x)
    pltpu.sync_copy(ox_hbm.at[pl.ds(base, NCHUNK)], off_x)

    iot = lax.iota(jnp.int32, LANES)
    zero = jnp.zeros((LANES,), jnp.float32)
    accs = (zero, zero, zero, zero)

    for p in range(NPASS):
        copies = []
        for j in range(CPP):
            copies.append(pltpu.async_copy(
                t_hbm.at[idx_c.at[p * CPP + j]],
                rows_c.at[pl.ds(j * CHUNK, CHUNK)], sem))
            copies.append(pltpu.async_copy(
                t_hbm.at[idx_x.at[p * CPP + j]],
                rows_x.at[pl.ds(j * CHUNK, CHUNK)], sem))
        for cp in copies:
            cp.wait()

        for j in range(CPP):
            for h in range(GPC):
                rvec = j * CHUNK + h * LANES + iot
                ocv = off_c[p * CPP + j, pl.ds(h * LANES, LANES)]
                oxv = off_x[p * CPP + j, pl.ds(h * LANES, LANES)]

                def body(k4, a, rvec=rvec, ocv=ocv, oxv=oxv):
                    out = []
                    for u in range(4):
                        k = k4 * 4 + u
                        cv = plsc.load_gather(rows_c, [rvec, ocv + k])
                        xv = plsc.load_gather(rows_x, [rvec, oxv + k])
                        out.append(a[u] + cv * xv)
                    return tuple(out)

                accs = lax.fori_loop(0, DIM // 4, body, accs)

    outv[...] = (accs[0] + accs[1]) + (accs[2] + accs[3])
    pltpu.sync_copy(outv, out_hbm.at[wid])


_sc_call = functools.partial(
    pl.kernel,
    mesh=plsc.VectorSubcoreMesh(core_axis_name="c", subcore_axis_name="s"),
    out_type=jax.ShapeDtypeStruct((NW, LANES), jnp.float32),
    scratch_types=[
        pltpu.VMEM((NCHUNK, CHUNK), jnp.int32),
        pltpu.VMEM((NCHUNK, CHUNK), jnp.int32),
        pltpu.VMEM((NCHUNK, CHUNK), jnp.int32),
        pltpu.VMEM((NCHUNK, CHUNK), jnp.int32),
        pltpu.VMEM((PP, CHUNK), jnp.float32),
        pltpu.VMEM((PP, CHUNK), jnp.float32),
        pltpu.VMEM((LANES,), jnp.float32),
        pltpu.SemaphoreType.DMA,
    ],
    compiler_params=pltpu.CompilerParams(
        use_tc_tiling_on_sc=True, needs_layout_passes=False),
)(_sc_body)


@jax.jit
def kernel(center_words, context_words, embeddings):
    table = _repack(embeddings.T)
    cw = jnp.asarray(center_words, jnp.int32)
    xw = jnp.asarray(context_words, jnp.int32)
    shape2 = (NW * NCHUNK, CHUNK)
    cb_bits = CB.bit_length() - 1

    def sup(r):
        return ((r >> cb_bits) << (cb_bits - 1)) | (r & (HB - 1))

    def off(r):
        return ((r >> (cb_bits - 1)) & 1) << 6

    gc = sup(cw).reshape(shape2)
    oc = off(cw).reshape(shape2)
    gx = sup(xw).reshape(shape2)
    ox = off(xw).reshape(shape2)
    partials = _sc_call(gc, oc, gx, ox, table)
    return jnp.sum(partials)


# speedup vs baseline: 2.1380x; 1.2182x over previous
import functools

import jax
import jax.numpy as jnp
from jax import lax
from jax.experimental import pallas as pl
from jax.experimental.pallas import tpu as pltpu
from jax.experimental.pallas import tpu_sc as plsc

VOCAB_ROWS = 1000000
DIM = 64
B = 16384
NC = 2
NS = 16
NW = NC * NS
BPW = B // NW
CHUNK = 128
NCHUNK = BPW // CHUNK
NPASS = 2
CPP = NCHUNK // NPASS
PP = BPW // NPASS
LANES = 16
GPC = CHUNK // LANES

CB = 8192
HB = CB // 2
NBLK = (VOCAB_ROWS + CB - 1) // CB
SROWS = NBLK * HB


def _repack_body(t_ref, out_ref):
    tt = t_ref[...].T
    out_ref[:, 0:DIM] = tt[0:HB]
    out_ref[:, DIM:2 * DIM] = tt[HB:CB]


_repack = pl.pallas_call(
    _repack_body,
    grid=(NBLK,),
    in_specs=[pl.BlockSpec((DIM, CB), lambda i: (0, i))],
    out_specs=pl.BlockSpec((HB, 128), lambda i: (i, 0)),
    out_shape=jax.ShapeDtypeStruct((SROWS, 128), jnp.float32),
)


def _sc_body(gc_hbm, oc_hbm, gx_hbm, ox_hbm, t_hbm, out_hbm,
             idx_c, idx_x, off_c, off_x, rows_c, rows_x, outv, sem):
    c = lax.axis_index("c")
    s = lax.axis_index("s")
    wid = s * NC + c
    base = wid * NCHUNK

    pltpu.sync_copy(gc_hbm.at[pl.ds(base, NCHUNK)], idx_c)
    pltpu.sync_copy(oc_hbm.at[pl.ds(base, NCHUNK)], off_c)
    pltpu.sync_copy(gx_hbm.at[pl.ds(base, NCHUNK)], idx_x)
    pltpu.sync_copy(ox_hbm.at[pl.ds(base, NCHUNK)], off_x)

    iot = lax.iota(jnp.int32, LANES)
    zero = jnp.zeros((LANES,), jnp.float32)
    accs = (zero, zero, zero, zero)

    for p in range(NPASS):
        copies = []
        for j in range(CPP):
            copies.append(pltpu.async_copy(
                t_hbm.at[idx_c.at[p * CPP + j]],
                rows_c.at[pl.ds(j * CHUNK, CHUNK)], sem))
            copies.append(pltpu.async_copy(
                t_hbm.at[idx_x.at[p * CPP + j]],
                rows_x.at[pl.ds(j * CHUNK, CHUNK)], sem))
        for cp in copies:
            cp.wait()

        for j in range(CPP):
            for h in range(GPC):
                rvec = j * CHUNK + h * LANES + iot
                ocv = off_c[p * CPP + j, pl.ds(h * LANES, LANES)]
                oxv = off_x[p * CPP + j, pl.ds(h * LANES, LANES)]

                def body(k4, a, rvec=rvec, ocv=ocv, oxv=oxv):
                    out = []
                    for u in range(4):
                        k = k4 * 4 + u
                        cv = plsc.load_gather(rows_c, [rvec, ocv + k])
                        xv = plsc.load_gather(rows_x, [rvec, oxv + k])
                        out.append(a[u] + cv * xv)
                    return tuple(out)

                accs = lax.fori_loop(0, DIM // 4, body, accs)

    outv[...] = (accs[0] + accs[1]) + (accs[2] + accs[3])
    pltpu.sync_copy(outv, out_hbm.at[wid])


_sc_call = functools.partial(
    pl.kernel,
    mesh=plsc.VectorSubcoreMesh(core_axis_name="c", subcore_axis_name="s"),
    out_type=jax.ShapeDtypeStruct((NW, LANES), jnp.float32),
    scratch_types=[
        pltpu.VMEM((NCHUNK, CHUNK), jnp.int32),
        pltpu.VMEM((NCHUNK, CHUNK), jnp.int32),
        pltpu.VMEM((NCHUNK, CHUNK), jnp.int32),
        pltpu.VMEM((NCHUNK, CHUNK), jnp.int32),
        pltpu.VMEM((PP, CHUNK), jnp.float32),
        pltpu.VMEM((PP, CHUNK), jnp.float32),
        pltpu.VMEM((LANES,), jnp.float32),
        pltpu.SemaphoreType.DMA,
    ],
    compiler_params=pltpu.CompilerParams(
        use_tc_tiling_on_sc=True, needs_layout_passes=False),
)(_sc_body)


@jax.jit
def kernel(center_words, context_words, embeddings):
    table = _repack(embeddings.T)
    cw = jnp.asarray(center_words, jnp.int32)
    xw = jnp.asarray(context_words, jnp.int32)
    shape2 = (NW * NCHUNK, CHUNK)
    cb_bits = CB.bit_length() - 1

    def sup(r):
        return ((r >> cb_bits) << (cb_bits - 1)) | (r & (HB - 1))

    def off(r):
        return ((r >> (cb_bits - 1)) & 1) << 6

    gc = sup(cw).reshape(shape2)
    oc = off(cw).reshape(shape2)
    gx = sup(xw).reshape(shape2)
    ox = off(xw).reshape(shape2)
    partials = _sc_call(gc, oc, gx, ox, table)
    return jnp.sum(partials)


# speedup vs baseline: 2.3832x; 1.1147x over previous
import functools

import jax
import jax.numpy as jnp
from jax import lax
from jax.experimental import pallas as pl
from jax.experimental.pallas import tpu as pltpu
from jax.experimental.pallas import tpu_sc as plsc

VOCAB_ROWS = 1000000
DIM = 64
B = 16384
NC = 2
NS = 16
NW = NC * NS
BPW = B // NW
CHUNK = 128
NCHUNK = BPW // CHUNK
NPASS = 2
CPP = NCHUNK // NPASS
PP = BPW // NPASS
LANES = 16
GPC = CHUNK // LANES

CB = 16384
HB = CB // 2
NBLK = (VOCAB_ROWS + CB - 1) // CB
SROWS = NBLK * HB


def _repack_body(t_ref, out_ref):
    tt = t_ref[...].T
    out_ref[:, 0:DIM] = tt[0:HB]
    out_ref[:, DIM:2 * DIM] = tt[HB:CB]


_repack = pl.pallas_call(
    _repack_body,
    grid=(NBLK,),
    in_specs=[pl.BlockSpec((DIM, CB), lambda i: (0, i))],
    out_specs=pl.BlockSpec((HB, 128), lambda i: (i, 0)),
    out_shape=jax.ShapeDtypeStruct((SROWS, 128), jnp.float32),
)


def _sc_body(gc_hbm, oc_hbm, gx_hbm, ox_hbm, t_hbm, out_hbm,
             idx_c, idx_x, off_c, off_x, rows_c, rows_x, outv, sem):
    c = lax.axis_index("c")
    s = lax.axis_index("s")
    wid = s * NC + c
    base = wid * NCHUNK

    pltpu.sync_copy(gc_hbm.at[pl.ds(base, NCHUNK)], idx_c)
    pltpu.sync_copy(oc_hbm.at[pl.ds(base, NCHUNK)], off_c)
    pltpu.sync_copy(gx_hbm.at[pl.ds(base, NCHUNK)], idx_x)
    pltpu.sync_copy(ox_hbm.at[pl.ds(base, NCHUNK)], off_x)

    iot = lax.iota(jnp.int32, LANES)
    zero = jnp.zeros((LANES,), jnp.float32)
    accs = (zero, zero, zero, zero)

    for p in range(NPASS):
        copies = []
        for j in range(CPP):
            copies.append(pltpu.async_copy(
                t_hbm.at[idx_c.at[p * CPP + j]],
                rows_c.at[pl.ds(j * CHUNK, CHUNK)], sem))
            copies.append(pltpu.async_copy(
                t_hbm.at[idx_x.at[p * CPP + j]],
                rows_x.at[pl.ds(j * CHUNK, CHUNK)], sem))
        for cp in copies:
            cp.wait()

        for j in range(CPP):
            for h in range(GPC):
                rvec = j * CHUNK + h * LANES + iot
                ocv = off_c[p * CPP + j, pl.ds(h * LANES, LANES)]
                oxv = off_x[p * CPP + j, pl.ds(h * LANES, LANES)]

                def body(k4, a, rvec=rvec, ocv=ocv, oxv=oxv):
                    out = []
                    for u in range(4):
                        k = k4 * 4 + u
                        cv = plsc.load_gather(rows_c, [rvec, ocv + k])
                        xv = plsc.load_gather(rows_x, [rvec, oxv + k])
                        out.append(a[u] + cv * xv)
                    return tuple(out)

                accs = lax.fori_loop(0, DIM // 4, body, accs)

    outv[...] = (accs[0] + accs[1]) + (accs[2] + accs[3])
    pltpu.sync_copy(outv, out_hbm.at[wid])


_sc_call = functools.partial(
    pl.kernel,
    mesh=plsc.VectorSubcoreMesh(core_axis_name="c", subcore_axis_name="s"),
    out_type=jax.ShapeDtypeStruct((NW, LANES), jnp.float32),
    scratch_types=[
        pltpu.VMEM((NCHUNK, CHUNK), jnp.int32),
        pltpu.VMEM((NCHUNK, CHUNK), jnp.int32),
        pltpu.VMEM((NCHUNK, CHUNK), jnp.int32),
        pltpu.VMEM((NCHUNK, CHUNK), jnp.int32),
        pltpu.VMEM((PP, CHUNK), jnp.float32),
        pltpu.VMEM((PP, CHUNK), jnp.float32),
        pltpu.VMEM((LANES,), jnp.float32),
        pltpu.SemaphoreType.DMA,
    ],
    compiler_params=pltpu.CompilerParams(
        use_tc_tiling_on_sc=True, needs_layout_passes=False),
)(_sc_body)


@jax.jit
def kernel(center_words, context_words, embeddings):
    table = _repack(embeddings.T)
    cw = jnp.asarray(center_words, jnp.int32)
    xw = jnp.asarray(context_words, jnp.int32)
    shape2 = (NW * NCHUNK, CHUNK)
    cb_bits = CB.bit_length() - 1

    def sup(r):
        return ((r >> cb_bits) << (cb_bits - 1)) | (r & (HB - 1))

    def off(r):
        return ((r >> (cb_bits - 1)) & 1) << 6

    gc = sup(cw).reshape(shape2)
    oc = off(cw).reshape(shape2)
    gx = sup(xw).reshape(shape2)
    ox = off(xw).reshape(shape2)
    partials = _sc_call(gc, oc, gx, ox, table)
    return jnp.sum(partials)
